# baseline (speedup 1.0000x reference)
import jax
import jax.numpy as jnp
from jax.experimental import pallas as pl
from jax.experimental.pallas import tpu as pltpu

N = 4096
D_IN = 256
D_HID = 256
D_OUT = 128
D_CAT = D_HID + D_IN
BI = 256
NB = N // BI
BO = 1024
NO = N // BO
EPS = 1e-8


def _feats_body(feat_ref, w1_ref, wa_ref, h1_ref, h2_ref):
    f = feat_ref[...]
    h1_ref[...] = jnp.dot(f, w1_ref[...], preferred_element_type=jnp.float32).astype(jnp.bfloat16)
    h2_ref[...] = jnp.dot(f, wa_ref[...], preferred_element_type=jnp.float32).astype(jnp.bfloat16)


def _main_body(dist_ref, cos_ref, h1_ref, h2_ref, b1_ref, b2_ref, wo_ref, bo_ref,
               out_ref, dn_ref, y_ref):
    i = pl.program_id(0)

    @pl.when(i < NB)
    def _agg():
        c = cos_ref[...]
        x2 = jnp.dot(c.astype(jnp.bfloat16), h2_ref[...], preferred_element_type=jnp.float32)
        d = dist_ref[...]
        inv_d = 1.0 / (jnp.sum(d, axis=1, keepdims=True) + EPS)
        inv_c = 1.0 / (jnp.sum(jnp.abs(c), axis=1, keepdims=True) + EPS)
        dn = (d * inv_d).astype(jnp.bfloat16)
        dn_ref[pl.ds(i * BI, BI), :] = dn
        x1 = jnp.dot(dn, h1_ref[...], preferred_element_type=jnp.float32)
        x1 = jnp.maximum(x1 + b1_ref[...], 0.0)
        x2 = jnp.maximum(x2 * inv_c + b2_ref[...], 0.0)
        yblk = (
            jnp.dot(x1, wo_ref[0:D_HID, :], preferred_element_type=jnp.float32)
            + jnp.dot(x2, wo_ref[D_HID:D_CAT, :], preferred_element_type=jnp.float32)
        )
        y_ref[pl.ds(i * BI, BI), :] = yblk.astype(jnp.bfloat16)

    @pl.when(i >= NB)
    def _out():
        j = i - NB
        dn = dn_ref[pl.ds(j * BO, BO), :]
        acc = jnp.dot(dn, y_ref[...], preferred_element_type=jnp.float32)
        out_ref[...] = acc + bo_ref[...]


def kernel(features, dist, adj_relative_cos, W_conv1, b_conv1, W_agg1, b_agg1, W_conv2, b_conv2):
    full = lambda i: (0, 0)

    h1, h2 = pl.pallas_call(
        _feats_body,
        out_shape=(
            jax.ShapeDtypeStruct((N, D_HID), jnp.bfloat16),
            jax.ShapeDtypeStruct((N, D_IN), jnp.bfloat16),
        ),
    )(features, W_conv1, W_agg1)

    adj_idx = lambda i: (jnp.clip(i, 0, NB - 1), 0)
    out_idx = lambda i: (jnp.maximum(i - NB, 0), 0)

    out = pl.pallas_call(
        _main_body,
        grid=(NB + NO,),
        in_specs=[
            pl.BlockSpec((BI, N), adj_idx),
            pl.BlockSpec((BI, N), adj_idx),
            pl.BlockSpec((N, D_HID), full),
            pl.BlockSpec((N, D_IN), full),
            pl.BlockSpec((1, D_HID), full),
            pl.BlockSpec((1, D_IN), full),
            pl.BlockSpec((D_CAT, D_OUT), full),
            pl.BlockSpec((1, D_OUT), full),
        ],
        out_specs=pl.BlockSpec((BO, D_OUT), out_idx),
        out_shape=jax.ShapeDtypeStruct((N, D_OUT), jnp.float32),
        scratch_shapes=[
            pltpu.VMEM((N, N), jnp.bfloat16),
            pltpu.VMEM((N, D_OUT), jnp.bfloat16),
        ],
    )(dist, adj_relative_cos, h1, h2,
      b_conv1.reshape(1, D_HID), b_agg1.reshape(1, D_IN),
      W_conv2, b_conv2.reshape(1, D_OUT))

    return out


# speedup vs baseline: 1.0061x; 1.0061x over previous
import jax
import jax.numpy as jnp
from jax.experimental import pallas as pl
from jax.experimental.pallas import tpu as pltpu

N = 4096
D_IN = 256
D_HID = 256
D_OUT = 128
D_CAT = D_HID + D_IN
BI = 256
NB = N // BI
BO = 2048
NO = N // BO
EPS = 1e-8


def _feats_body(feat_ref, w1_ref, wa_ref, h1_ref, h2_ref):
    f = feat_ref[...]
    h1_ref[...] = jnp.dot(f, w1_ref[...], preferred_element_type=jnp.float32).astype(jnp.bfloat16)
    h2_ref[...] = jnp.dot(f, wa_ref[...], preferred_element_type=jnp.float32).astype(jnp.bfloat16)


def _main_body(dist_ref, cos_ref, h1_ref, h2_ref, b1_ref, b2_ref, wo_ref, bo_ref,
               out_ref, dn_ref, y_ref):
    i = pl.program_id(0)

    @pl.when(i < NB)
    def _agg():
        c = cos_ref[...]
        x2 = jnp.dot(c.astype(jnp.bfloat16), h2_ref[...], preferred_element_type=jnp.float32)
        d = dist_ref[...]
        inv_d = 1.0 / (jnp.sum(d, axis=1, keepdims=True) + EPS)
        inv_c = 1.0 / (jnp.sum(jnp.abs(c), axis=1, keepdims=True) + EPS)
        dn = (d * inv_d).astype(jnp.bfloat16)
        dn_ref[pl.ds(i * BI, BI), :] = dn
        x1 = jnp.dot(dn, h1_ref[...], preferred_element_type=jnp.float32)
        x1 = jnp.maximum(x1 + b1_ref[...], 0.0)
        x2 = jnp.maximum(x2 * inv_c + b2_ref[...], 0.0)
        yblk = (
            jnp.dot(x1, wo_ref[0:D_HID, :], preferred_element_type=jnp.float32)
            + jnp.dot(x2, wo_ref[D_HID:D_CAT, :], preferred_element_type=jnp.float32)
        )
        y_ref[pl.ds(i * BI, BI), :] = yblk.astype(jnp.bfloat16)

    @pl.when(i >= NB)
    def _out():
        j = i - NB
        dn = dn_ref[pl.ds(j * BO, BO), :]
        acc = jnp.dot(dn, y_ref[...], preferred_element_type=jnp.float32)
        out_ref[...] = acc + bo_ref[...]


def kernel(features, dist, adj_relative_cos, W_conv1, b_conv1, W_agg1, b_agg1, W_conv2, b_conv2):
    full = lambda i: (0, 0)

    h1, h2 = pl.pallas_call(
        _feats_body,
        out_shape=(
            jax.ShapeDtypeStruct((N, D_HID), jnp.bfloat16),
            jax.ShapeDtypeStruct((N, D_IN), jnp.bfloat16),
        ),
    )(features, W_conv1, W_agg1)

    adj_idx = lambda i: (jnp.clip(i, 0, NB - 1), 0)
    out_idx = lambda i: (jnp.maximum(i - NB, 0), 0)

    out = pl.pallas_call(
        _main_body,
        grid=(NB + NO,),
        in_specs=[
            pl.BlockSpec((BI, N), adj_idx),
            pl.BlockSpec((BI, N), adj_idx),
            pl.BlockSpec((N, D_HID), full),
            pl.BlockSpec((N, D_IN), full),
            pl.BlockSpec((1, D_HID), full),
            pl.BlockSpec((1, D_IN), full),
            pl.BlockSpec((D_CAT, D_OUT), full),
            pl.BlockSpec((1, D_OUT), full),
        ],
        compiler_params=pltpu.CompilerParams(vmem_limit_bytes=63 * 1024 * 1024),
        out_specs=pl.BlockSpec((BO, D_OUT), out_idx),
        out_shape=jax.ShapeDtypeStruct((N, D_OUT), jnp.float32),
        scratch_shapes=[
            pltpu.VMEM((N, N), jnp.bfloat16),
            pltpu.VMEM((N, D_OUT), jnp.bfloat16),
        ],
    )(dist, adj_relative_cos, h1, h2,
      b_conv1.reshape(1, D_HID), b_agg1.reshape(1, D_IN),
      W_conv2, b_conv2.reshape(1, D_OUT))

    return out


# speedup vs baseline: 1.0314x; 1.0251x over previous
import jax
import jax.numpy as jnp
from jax.experimental import pallas as pl
from jax.experimental.pallas import tpu as pltpu

N = 4096
D_IN = 256
D_HID = 256
D_OUT = 128
D_CAT = D_HID + D_IN
BI = 256
NB = N // BI
BO = 512
NO = N // BO
EPS = 1e-8


def _main_body(dist_ref, cos_ref, feat_ref, w1_ref, wa_ref, b1_ref, b2_ref,
               wo_ref, bo_ref, out_ref, h1_ref, h2_ref, dn_ref, y_ref):
    i = pl.program_id(0)

    @pl.when(i == 0)
    def _feats():
        f = feat_ref[...]
        h1_ref[...] = jnp.dot(f, w1_ref[...], preferred_element_type=jnp.float32).astype(jnp.bfloat16)
        h2_ref[...] = jnp.dot(f, wa_ref[...], preferred_element_type=jnp.float32).astype(jnp.bfloat16)

    @pl.when((i >= 1) & (i <= NB))
    def _agg():
        j = i - 1
        c = cos_ref[...]
        x2 = jnp.dot(c.astype(jnp.bfloat16), h2_ref[...], preferred_element_type=jnp.float32)
        d = dist_ref[...]
        inv_d = 1.0 / (jnp.sum(d, axis=1, keepdims=True) + EPS)
        inv_c = 1.0 / (jnp.sum(jnp.abs(c), axis=1, keepdims=True) + EPS)
        dn = (d * inv_d).astype(jnp.bfloat16)
        dn_ref[pl.ds(j * BI, BI), :] = dn
        x1 = jnp.dot(dn, h1_ref[...], preferred_element_type=jnp.float32)
        x1 = jnp.maximum(x1 + b1_ref[...], 0.0)
        x2 = jnp.maximum(x2 * inv_c + b2_ref[...], 0.0)
        yblk = (
            jnp.dot(x1, wo_ref[0:D_HID, :], preferred_element_type=jnp.float32)
            + jnp.dot(x2, wo_ref[D_HID:D_CAT, :], preferred_element_type=jnp.float32)
        )
        y_ref[pl.ds(j * BI, BI), :] = yblk.astype(jnp.bfloat16)

    @pl.when(i > NB)
    def _out():
        j = i - (NB + 1)
        dn = dn_ref[pl.ds(j * BO, BO), :]
        acc = jnp.dot(dn, y_ref[...], preferred_element_type=jnp.float32)
        out_ref[...] = acc + bo_ref[...]


def kernel(features, dist, adj_relative_cos, W_conv1, b_conv1, W_agg1, b_agg1, W_conv2, b_conv2):
    full = lambda i: (0, 0)

    adj_idx = lambda i: (jnp.clip(i - 1, 0, NB - 1), 0)
    out_idx = lambda i: (jnp.clip(i - (NB + 1), 0, NO - 1), 0)

    out = pl.pallas_call(
        _main_body,
        grid=(1 + NB + NO,),
        in_specs=[
            pl.BlockSpec((BI, N), adj_idx),
            pl.BlockSpec((BI, N), adj_idx),
            pl.BlockSpec((N, D_IN), full),
            pl.BlockSpec((D_IN, D_HID), full),
            pl.BlockSpec((D_IN, D_IN), full),
            pl.BlockSpec((1, D_HID), full),
            pl.BlockSpec((1, D_IN), full),
            pl.BlockSpec((D_CAT, D_OUT), full),
            pl.BlockSpec((1, D_OUT), full),
        ],
        compiler_params=pltpu.CompilerParams(vmem_limit_bytes=63 * 1024 * 1024),
        out_specs=pl.BlockSpec((BO, D_OUT), out_idx),
        out_shape=jax.ShapeDtypeStruct((N, D_OUT), jnp.float32),
        scratch_shapes=[
            pltpu.VMEM((N, D_HID), jnp.bfloat16),
            pltpu.VMEM((N, D_IN), jnp.bfloat16),
            pltpu.VMEM((N, N), jnp.bfloat16),
            pltpu.VMEM((N, D_OUT), jnp.bfloat16),
        ],
    )(dist, adj_relative_cos, features, W_conv1, W_agg1,
      b_conv1.reshape(1, D_HID), b_agg1.reshape(1, D_IN),
      W_conv2, b_conv2.reshape(1, D_OUT))

    return out
